# Initial kernel scaffold; baseline (speedup 1.0000x reference)
import jax
import jax.numpy as jnp
from jax.experimental import pallas as pl


def kernel(tX, tB):
    raise NotImplementedError("write your pallas kernel here")


# speedup vs baseline: 7.2061x; 7.2061x over previous
import functools

import jax
import jax.numpy as jnp
from jax import lax
from jax.experimental import pallas as pl
from jax.experimental.pallas import tpu as pltpu
from jax.experimental.pallas import tpu_sc as plsc

GS = 64
CELLS = GS * GS
L = 16
NT = 16
CS = CELLS // NT


def _shift_up(x):
    idx = jnp.minimum(lax.iota(jnp.int32, L) + 1, L - 1)
    dn = lax.GatherDimensionNumbers(
        offset_dims=(), collapsed_slice_dims=(0,), start_index_map=(0,))
    return lax.gather(x, idx[:, None], dn, (1,),
                      mode=lax.GatherScatterMode.PROMISE_IN_BOUNDS)


def _make_grid_kernel(nboxes: int):
    cnt_max = -(-nboxes // NT)
    cnt_max = -(-cnt_max // 8) * 8
    vpt = -(-cnt_max // L)
    padn = (NT - 1) * cnt_max + vpt * L

    mesh = plsc.VectorSubcoreMesh(
        core_axis_name="c", subcore_axis_name="s", num_cores=1)

    @functools.partial(
        pl.kernel,
        out_type=jax.ShapeDtypeStruct((6, CELLS), jnp.float32),
        mesh=mesh,
        compiler_params=pltpu.CompilerParams(needs_layout_passes=False),
        scratch_types=[
            pltpu.VMEM((vpt * L,), jnp.float32),
            pltpu.VMEM((vpt * L,), jnp.float32),
            pltpu.VMEM((CELLS,), jnp.int32),
            pltpu.VMEM_SHARED((NT, CELLS), jnp.int32),
            pltpu.VMEM((NT, CS), jnp.int32),
            pltpu.VMEM((CS,), jnp.int32),
            pltpu.VMEM((2, 128), jnp.int32),
            pltpu.VMEM((CS,), jnp.float32),
            pltpu.VMEM((CS,), jnp.float32),
            pltpu.VMEM((CS,), jnp.float32),
            pltpu.VMEM((CS,), jnp.float32),
            pltpu.VMEM((CS,), jnp.float32),
            pltpu.VMEM((6, CS), jnp.float32),
            pltpu.SemaphoreType.DMA,
        ],
    )
    def grid_kernel(xh, yh, wh, hh, clh, outh,
                    xv, yv, grid, shared, tmp, wv, idx2,
                    gx, gy, gw, gh, gc, outb, sem):
        sid = lax.axis_index("s")
        base = sid * cnt_max
        cnt = jnp.minimum(jnp.int32(cnt_max), jnp.int32(nboxes) - base)

        cpx = pltpu.async_copy(xh.at[pl.ds(base, vpt * L)], xv, sem)
        cpy = pltpu.async_copy(yh.at[pl.ds(base, vpt * L)], yv, sem)

        neg1 = jnp.full((L,), -1, jnp.int32)

        def init_body(i, carry):
            grid[pl.ds(i * L, L)] = neg1
            return carry

        lax.fori_loop(0, CELLS // L, init_body, 0)
        cpx.wait()
        cpy.wait()

        lane = lax.iota(jnp.int32, L)

        def box_body(v, carry):
            lx = xv[pl.ds(v * L, L)]
            ly = yv[pl.ds(v * L, L)]
            cx = (lx * 64.0).astype(jnp.int32)
            cy = (ly * 64.0).astype(jnp.int32)
            cell = cy * GS + cx
            local = v * L + lane
            valid = local < cnt
            cellk = jnp.where(valid, cell, CELLS + lane)
            key = cellk * L + lane
            skey, sloc = plsc.sort_key_val(key, local)
            scell = skey >> 4
            nxt = _shift_up(scell)
            keep = (scell != nxt) | (lane == L - 1)
            m = keep & (sloc < cnt)
            plsc.store_scatter(grid, [jnp.minimum(scell, CELLS - 1)],
                               sloc + base, mask=m)
            return carry

        lax.fori_loop(0, vpt, box_body, 0)

        pltpu.sync_copy(grid, shared.at[sid])
        plsc.subcore_barrier()
        cbase = sid * CS
        for g in range(NT):
            pltpu.sync_copy(shared.at[g, pl.ds(cbase, CS)], tmp.at[g])
        for j in range(CS // L):
            w = tmp[0, pl.ds(j * L, L)]
            for g in range(1, NT):
                w = jnp.maximum(w, tmp[g, pl.ds(j * L, L)])
            wv[pl.ds(j * L, L)] = w

        for c in range(2):
            for j in range(128 // L):
                wvec = wv[pl.ds(c * 128 + j * L, L)]
                idx2[c, pl.ds(j * L, L)] = jnp.maximum(wvec, 0)
        for c in range(2):
            dst = pl.ds(c * 128, 128)
            pltpu.async_copy(xh.at[idx2.at[c]], gx.at[dst], sem).wait()
            pltpu.async_copy(yh.at[idx2.at[c]], gy.at[dst], sem).wait()
            pltpu.async_copy(wh.at[idx2.at[c]], gw.at[dst], sem).wait()
            pltpu.async_copy(hh.at[idx2.at[c]], gh.at[dst], sem).wait()
            pltpu.async_copy(clh.at[idx2.at[c]], gc.at[dst], sem).wait()

        zero = jnp.zeros((L,), jnp.float32)
        one = jnp.ones((L,), jnp.float32)
        for j in range(CS // L):
            sl = pl.ds(j * L, L)
            has = wv[sl] >= 0
            px = gx[sl] * 64.0
            py = gy[sl] * 64.0
            fx = px - px.astype(jnp.int32).astype(jnp.float32)
            fy = py - py.astype(jnp.int32).astype(jnp.float32)
            outb[0, sl] = jnp.where(has, one, zero)
            outb[1, sl] = jnp.where(has, fx, zero)
            outb[2, sl] = jnp.where(has, fy, zero)
            outb[3, sl] = jnp.where(has, gw[sl] * 64.0, zero)
            outb[4, sl] = jnp.where(has, gh[sl] * 64.0, zero)
            outb[5, sl] = jnp.where(has, gc[sl], zero)
        for ch in range(6):
            pltpu.sync_copy(outb.at[ch], outh.at[ch, pl.ds(cbase, CS)])

    return grid_kernel, padn


def kernel(tX, tB):
    nboxes = tB.shape[0]
    gk, padn = _make_grid_kernel(nboxes)
    cols = tB.T
    colsp = jnp.pad(cols, ((0, 0), (0, padn - nboxes)))
    tY = gk(colsp[1], colsp[2], colsp[3], colsp[4], colsp[0])
    return (tX, tY.reshape(6, GS, GS))
